# baseline (speedup 1.0000x reference)
import jax
import jax.numpy as jnp
from jax import lax
from jax.experimental import pallas as pl
from jax.experimental.pallas import tpu as pltpu
from jax.experimental.pallas import tpu_sc as plsc

_NUM_ANCHORS = 5
_ANCHORS = [0.57273, 0.677385, 1.87446, 2.06253, 3.33843, 5.47434,
            7.88282, 3.52778, 9.77052, 9.16828]
_CONF_THRES = 0.6
_IOU_THRES = 0.4
_H = 19
_W = 19
_HW = _H * _W
_N = _NUM_ANCHORS * _HW
_N2 = 2048
_L = 16
_NCHUNK = _N2 // _L
_NWORD = _NCHUNK // _L


def _sig(v):
    return 1.0 / (1.0 + jnp.exp(-v))


def _vsum(v):
    return lax.rev(plsc.cumsum(v), (0,))[0]


def _vlane(v, l):
    return _vsum(jnp.where(lax.iota(v.dtype if v.dtype == jnp.int32 else jnp.int32, 16) == l, v, jnp.zeros_like(v)))


def _sc_body(in_hbm, out_hbm, in_v, cx1, cx2, cy1, cy2, car, cc,
             m_ref, alive_f, aw_ref, tw_ref, sw_ref, gbuf, shared, out_stage):
    f32 = jnp.float32
    i32 = jnp.int32
    cid = lax.axis_index("c")
    sid = lax.axis_index("s")
    iota = lax.iota(i32, _L)
    one = jnp.full((_L,), 1, i32)

    pltpu.sync_copy(in_hbm, in_v)

    zf = jnp.zeros((_L,), f32)

    def zero_body(c, _):
        off = c * _L
        for ref in (cx1, cx2, cy1, cy2, car, cc):
            ref[pl.ds(off, _L)] = zf
        return 0

    lax.fori_loop(0, (_N2 + _L) // _L, zero_body, 0)

    def dc_body(c, off):
        base = c * _L
        idx = base + iota
        rx = in_v[0, pl.ds(base, _L)]
        ry = in_v[1, pl.ds(base, _L)]
        rw = in_v[2, pl.ds(base, _L)]
        rh = in_v[3, pl.ds(base, _L)]
        rc = in_v[4, pl.ds(base, _L)]
        cell = idx % _HW
        a = idx // _HW
        gx = (cell % _W).astype(f32)
        gy = (cell // _W).astype(f32)
        aw = jnp.full((_L,), _ANCHORS[8], f32)
        ah = jnp.full((_L,), _ANCHORS[9], f32)
        for k in range(_NUM_ANCHORS - 1):
            aw = jnp.where(a == k, _ANCHORS[2 * k], aw)
            ah = jnp.where(a == k, _ANCHORS[2 * k + 1], ah)
        x = (_sig(rx) + gx) * (1.0 / _W)
        y = (_sig(ry) + gy) * (1.0 / _H)
        w = jnp.exp(rw) * aw * (1.0 / _W)
        h = jnp.exp(rh) * ah * (1.0 / _H)
        det = _sig(rc)
        conf = jnp.where((det > _CONF_THRES) & (idx < _N), det, 0.0)
        m = conf > 0.0
        csum = plsc.cumsum(jnp.where(m, 1, 0).astype(i32))
        pos = off + csum - 1
        plsc.store_scatter(cx1, [pos], x - 0.5 * w, mask=m)
        plsc.store_scatter(cx2, [pos], x + 0.5 * w, mask=m)
        plsc.store_scatter(cy1, [pos], y - 0.5 * h, mask=m)
        plsc.store_scatter(cy2, [pos], y + 0.5 * h, mask=m)
        plsc.store_scatter(car, [pos], w * h, mask=m)
        plsc.store_scatter(cc, [pos], conf, mask=m)
        return off + lax.rev(csum, (0,))[0]

    kcnt = lax.fori_loop(0, _NCHUNK, dc_body, jnp.int32(0))

    nch = (kcnt + _L - 1) // _L
    nwrd = (nch + _L - 1) // _L
    nrow = jnp.maximum((kcnt - sid + 15) // 16, 0)

    def tw_body(w, _):
        cbase = (w * _L + iota) * _L
        nbits = jnp.clip(kcnt - cbase, 0, _L)
        word = jnp.left_shift(one, nbits) - 1
        tw_ref[pl.ds(w * _L, _L)] = word
        aw_ref[pl.ds(w * _L, _L)] = word
        sw_ref[pl.ds(w * _L, _L)] = jnp.zeros((_L,), i32)
        return 0

    lax.fori_loop(0, _NWORD, tw_body, 0)

    def af_body(c, _):
        idx = c * _L + iota
        alive_f[pl.ds(c * _L, _L)] = jnp.where(idx < kcnt, 1.0, 0.0)
        return 0

    lax.fori_loop(0, _NCHUNK, af_body, 0)

    def row_body(k, _):
        r = sid + k * 16
        x1i = cx1[pl.ds(r, _L)][0]
        x2i = cx2[pl.ds(r, _L)][0]
        y1i = cy1[pl.ds(r, _L)][0]
        y2i = cy2[pl.ds(r, _L)][0]
        ai = car[pl.ds(r, _L)][0]
        ci = cc[pl.ds(r, _L)][0]
        wi = x2i - x1i
        hi = y2i - y1i

        def w_body(w, _):
            def c_body(cw_, acc):
                c = w * _L + cw_
                colo = c * _L
                x1j = cx1[pl.ds(colo, _L)]
                x2j = cx2[pl.ds(colo, _L)]
                y1j = cy1[pl.ds(colo, _L)]
                y2j = cy2[pl.ds(colo, _L)]
                aj = car[pl.ds(colo, _L)]
                cj = cc[pl.ds(colo, _L)]
                uw = jnp.maximum(x2i, x2j) - jnp.minimum(x1i, x1j)
                uh = jnp.maximum(y2i, y2j) - jnp.minimum(y1i, y1j)
                cwv = (wi + (x2j - x1j)) - uw
                chv = (hi + (y2j - y1j)) - uh
                cav = cwv * chv
                uav = (ai + aj) - cav
                ov = (cwv > 0) & (chv > 0) & (cav > _IOU_THRES * uav)
                prec = (ci > cj) | ((ci == cj) & (r < colo + iota))
                mb = ov & prec & (cj > 0)
                bits = _vsum(jnp.where(mb, jnp.left_shift(one, iota),
                                     jnp.zeros((_L,), i32)))
                return acc | jnp.where(iota == cw_, bits, 0)

            acc = lax.fori_loop(0, _L, c_body, jnp.zeros((_L,), i32),
                                unroll=True)
            m_ref[k, pl.ds(w * _L, _L)] = acc
            sw_ref[pl.ds(w * _L, _L)] = sw_ref[pl.ds(w * _L, _L)] | acc
            return 0

        lax.fori_loop(0, nwrd, w_body, 0)
        return 0

    lax.fori_loop(0, nrow, row_body, 0)

    def cond(carry):
        it, changed = carry
        return changed & (it < _N2 + 1)

    def body(carry):
        it, _ = carry
        slot = lax.rem(it, 2)

        pltpu.sync_copy(sw_ref, shared.at[slot, sid])
        plsc.subcore_barrier()
        pltpu.sync_copy(shared.at[slot], gbuf)

        def orr(w, _):
            acc = gbuf[0, pl.ds(w * _L, _L)]
            for t_ in range(1, 16):
                acc = acc | gbuf[t_, pl.ds(w * _L, _L)]
            sw_ref[pl.ds(w * _L, _L)] = acc
            return 0

        lax.fori_loop(0, nwrd, orr, 0)

        def upd(w, nd):
            old = aw_ref[pl.ds(w * _L, _L)]
            new = tw_ref[pl.ds(w * _L, _L)] & ~sw_ref[pl.ds(w * _L, _L)]
            aw_ref[pl.ds(w * _L, _L)] = new
            return nd + _vsum(jnp.where(new != old, one, jnp.zeros((_L,), i32)))

        ndiff = lax.fori_loop(0, nwrd, upd, jnp.int32(0))

        def exp_body(c, _):
            w = c // _L
            l = c % _L
            word = _vlane(aw_ref[pl.ds(w * _L, _L)], l)
            alive_f[pl.ds(c * _L, _L)] = (
                jnp.right_shift(word, iota) & 1).astype(f32)
            return 0

        lax.fori_loop(0, nch, exp_body, 0)

        def zw(w, _):
            sw_ref[pl.ds(w * _L, _L)] = jnp.zeros((_L,), i32)
            return 0

        lax.fori_loop(0, _NWORD, zw, 0)

        def prow(k, _):
            r = sid + k * 16

            @pl.when(alive_f[pl.ds(r, _L)][0] > 0.0)
            def _():
                def pw(w, _):
                    sw_ref[pl.ds(w * _L, _L)] = (
                        sw_ref[pl.ds(w * _L, _L)]
                        | m_ref[k, pl.ds(w * _L, _L)])
                    return 0

                lax.fori_loop(0, nwrd, pw, 0)

            return 0

        lax.fori_loop(0, nrow, prow, 0)
        return it + 1, ndiff > 0

    lax.while_loop(cond, body, (jnp.int32(0), True))

    def fin(c, carry):
        tot, cnt = carry
        al = alive_f[pl.ds(c * _L, _L)]
        cv = cc[pl.ds(c * _L, _L)]
        return tot + _vsum(al * cv), cnt + _vsum(al)

    tot, cnt = lax.fori_loop(0, nch, fin,
                             (jnp.float32(0.0), jnp.float32(0.0)))
    totv = jnp.full((_L,), 1.0, f32) * tot
    cntv = jnp.full((_L,), 1.0, f32) * cnt
    meanv = totv / jnp.where(cntv > 0, cntv, jnp.full((_L,), 1.0, f32))
    meanv = jnp.where(cntv > 0, meanv, jnp.zeros((_L,), f32))

    @pl.when((cid == 0) & (sid == 0))
    def _():
        out_stage[...] = jnp.where(iota == 0, meanv, 0.0)
        pltpu.sync_copy(out_stage, out_hbm)


def kernel(output):
    raw = output[0].reshape(_NUM_ANCHORS, 5 + 80, _HW)[:, :5, :]
    rows = raw.transpose(1, 0, 2).reshape(5, _N)
    rows = jnp.pad(rows, ((0, 0), (0, _N2 - _N)))
    mesh = plsc.VectorSubcoreMesh(core_axis_name="c", subcore_axis_name="s",
                                  num_cores=1, num_subcores=16)
    sc = pl.kernel(
        _sc_body,
        out_type=jax.ShapeDtypeStruct((_L,), jnp.float32),
        mesh=mesh,
        compiler_params=pltpu.CompilerParams(needs_layout_passes=False),
        scratch_types=[
            pltpu.VMEM((5, _N2), jnp.float32),
            pltpu.VMEM((_N2 + _L,), jnp.float32),
            pltpu.VMEM((_N2 + _L,), jnp.float32),
            pltpu.VMEM((_N2 + _L,), jnp.float32),
            pltpu.VMEM((_N2 + _L,), jnp.float32),
            pltpu.VMEM((_N2 + _L,), jnp.float32),
            pltpu.VMEM((_N2 + _L,), jnp.float32),
            pltpu.VMEM((_NCHUNK, _NCHUNK), jnp.int32),
            pltpu.VMEM((_N2 + _L,), jnp.float32),
            pltpu.VMEM((_NCHUNK,), jnp.int32),
            pltpu.VMEM((_NCHUNK,), jnp.int32),
            pltpu.VMEM((_NCHUNK,), jnp.int32),
            pltpu.VMEM((16, _NCHUNK), jnp.int32),
            pltpu.VMEM_SHARED((2, 16, _NCHUNK), jnp.int32),
            pltpu.VMEM((_L,), jnp.float32),
        ],
    )
    res = sc(rows)
    return res[0].reshape(())
